# baseline (speedup 1.0000x reference)
import jax
import jax.numpy as jnp
from jax import lax
from jax.experimental import pallas as pl
from jax.experimental.pallas import tpu as pltpu
from jax.experimental.pallas import tpu_sc as plsc

R = 1000000
D = 32
B = 16384

NC = 2
NS = 16
L = 16
NW = NC * NS
BPW = B // NW


def _gmf_body(users_h, items_h, ut_h, it_h, w_h, b_h, out_h,
              uidx, iidx, gu, gi, wv, bv, outv, sem):
    wid = lax.axis_index("s") * NC + lax.axis_index("c")
    base = wid * BPW

    pltpu.sync_copy(users_h.at[pl.ds(base, BPW)], uidx)
    pltpu.sync_copy(items_h.at[pl.ds(base, BPW)], iidx)
    pltpu.sync_copy(w_h, wv)
    pltpu.sync_copy(b_h, bv)

    copies = []
    for d in range(D):
        copies.append(pltpu.async_copy(
            ut_h.at[pl.ds(d * R, R)].at[uidx], gu.at[pl.ds(d * BPW, BPW)], sem))
        copies.append(pltpu.async_copy(
            it_h.at[pl.ds(d * R, R)].at[iidx], gi.at[pl.ds(d * BPW, BPW)], sem))

    w0 = wv[pl.ds(0, L)]
    w1 = wv[pl.ds(L, L)]
    bias = bv[...]

    for c in copies:
        c.wait()

    def col(k, carry):
        off = k * L
        acc = bias
        for dd, wreg in ((0, w0), (1, w1)):
            for j in range(L):
                d = dd * L + j
                wd = lax.broadcast(wreg[j], (L,))
                u = gu[pl.ds(d * BPW + off, L)]
                i = gi[pl.ds(d * BPW + off, L)]
                acc = acc + wd * u * i
        o = 1.0 / (1.0 + jnp.exp(-acc))
        outv[pl.ds(off, L)] = o
        return carry

    lax.fori_loop(0, BPW // L, col, 0)

    pltpu.sync_copy(outv, out_h.at[pl.ds(base, BPW)])


@jax.jit
def _gmf(users, items, ut_flat, it_flat, w_flat, b_vec):
    mesh = plsc.VectorSubcoreMesh(core_axis_name="c", subcore_axis_name="s",
                                  num_cores=NC, num_subcores=NS)
    run = pl.kernel(
        _gmf_body,
        out_type=jax.ShapeDtypeStruct((B,), jnp.float32),
        mesh=mesh,
        compiler_params=pltpu.CompilerParams(needs_layout_passes=False),
        scratch_types=[
            pltpu.VMEM((BPW,), jnp.int32),
            pltpu.VMEM((BPW,), jnp.int32),
            pltpu.VMEM((D * BPW,), jnp.float32),
            pltpu.VMEM((D * BPW,), jnp.float32),
            pltpu.VMEM((D,), jnp.float32),
            pltpu.VMEM((L,), jnp.float32),
            pltpu.VMEM((BPW,), jnp.float32),
            pltpu.SemaphoreType.DMA,
        ],
    )
    return run(users, items, ut_flat, it_flat, w_flat, b_vec)


def kernel(items, users, user_table, item_table, W, b):
    ut_flat = user_table.T.reshape(R * D)
    it_flat = item_table.T.reshape(R * D)
    w_flat = W.reshape(D)
    b_vec = jnp.broadcast_to(b.reshape(()), (L,))
    out = _gmf(users.astype(jnp.int32), items.astype(jnp.int32),
               ut_flat, it_flat, w_flat, b_vec)
    return out.reshape(B, 1)
